# baseline (speedup 1.0000x reference)
import functools

import jax
import jax.numpy as jnp
from jax import lax
from jax.experimental import pallas as pl
from jax.experimental.pallas import tpu as pltpu
from jax.experimental.pallas import tpu_sc as plsc

D = 64
NC = 2
NS = 16
NW = NC * NS
GSUB = 128
NG = 4
CHUNK = GSUB * NG


@functools.partial(jax.jit, static_argnames=())
def _embed_lookup(x2d, weight):
    n_rows, _ = x2d.shape
    B = n_rows * GSUB
    b_per_w = B // NW
    n_chunks = b_per_w // CHUNK
    rows_per_chunk = CHUNK // GSUB

    mesh = plsc.VectorSubcoreMesh(core_axis_name="c", subcore_axis_name="s")

    @functools.partial(
        pl.kernel,
        mesh=mesh,
        out_type=jax.ShapeDtypeStruct((B, D), jnp.float32),
        compiler_params=pltpu.CompilerParams(use_tc_tiling_on_sc=False),
        scratch_types=[
            pltpu.VMEM((NG, GSUB), jnp.int32),
            pltpu.VMEM((CHUNK, D), jnp.float32),
            pltpu.SemaphoreType.DMA,
        ],
    )
    def k(idx_hbm, table_hbm, out_hbm, idx_v, rows_v, sem):
        wid = lax.axis_index("s") * NC + lax.axis_index("c")
        row_base = wid * (b_per_w // GSUB)
        out_base = wid * b_per_w

        def body(c, carry):
            pltpu.sync_copy(
                idx_hbm.at[pl.ds(row_base + c * rows_per_chunk, NG)], idx_v
            )
            copies = []
            for j in range(NG):
                copies.append(
                    pltpu.async_copy(
                        table_hbm.at[idx_v.at[j]],
                        rows_v.at[pl.ds(j * GSUB, GSUB)],
                        sem,
                    )
                )
            for cp in copies:
                cp.wait()
            pltpu.sync_copy(
                rows_v, out_hbm.at[pl.ds(out_base + c * CHUNK, CHUNK)]
            )
            return carry

        lax.fori_loop(0, n_chunks, body, 0)

    return k(x2d, weight)


def kernel(x, weight):
    B_, S_ = x.shape
    x2d = x.reshape(-1, GSUB).astype(jnp.int32)
    out = _embed_lookup(x2d, weight)
    return out.reshape(B_, S_, D)


# speedup vs baseline: 1.0694x; 1.0694x over previous
import functools

import jax
import jax.numpy as jnp
from jax import lax
from jax.experimental import pallas as pl
from jax.experimental.pallas import tpu as pltpu
from jax.experimental.pallas import tpu_sc as plsc

D = 64
NC = 2
NS = 16
NW = NC * NS
GSUB = 128
NG = 4
CHUNK = GSUB * NG
NBUF = 2


@jax.jit
def _embed_lookup(x2d, weight):
    n_rows, _ = x2d.shape
    B = n_rows * GSUB
    b_per_w = B // NW
    n_chunks = b_per_w // CHUNK
    n_pairs = n_chunks // NBUF

    mesh = plsc.VectorSubcoreMesh(core_axis_name="c", subcore_axis_name="s")

    @functools.partial(
        pl.kernel,
        mesh=mesh,
        out_type=jax.ShapeDtypeStruct((B, D), jnp.float32),
        compiler_params=pltpu.CompilerParams(use_tc_tiling_on_sc=False),
        scratch_types=[
            pltpu.VMEM((NBUF, NG, GSUB), jnp.int32),
            pltpu.VMEM((NBUF, CHUNK, D), jnp.float32),
            pltpu.SemaphoreType.DMA,
            pltpu.SemaphoreType.DMA,
            pltpu.SemaphoreType.DMA,
            pltpu.SemaphoreType.DMA,
            pltpu.SemaphoreType.DMA,
            pltpu.SemaphoreType.DMA,
        ],
    )
    def k(idx_hbm, table_hbm, out_hbm, idx_v, rows_v,
          isem0, isem1, gsem0, gsem1, osem0, osem1):
        isems = (isem0, isem1)
        gsems = (gsem0, gsem1)
        osems = (osem0, osem1)
        wid = lax.axis_index("s") * NC + lax.axis_index("c")
        row_base = wid * (b_per_w // GSUB)
        out_base = wid * b_per_w

        def idx_slice(c):
            return idx_hbm.at[pl.ds(row_base + c * NG, NG)]

        def out_slice(c):
            return out_hbm.at[pl.ds(out_base + c * CHUNK, CHUNK)]

        for b in range(NBUF):
            pltpu.async_copy(idx_slice(b), idx_v.at[b], isems[b])

        def pair_body(g, carry):
            descs = []
            for b in range(NBUF):
                c = g * NBUF + b

                @pl.when(g > 0)
                def _():
                    pltpu.make_async_copy(
                        rows_v.at[b], out_slice(c), osems[b]
                    ).wait()

                pltpu.make_async_copy(idx_slice(c), idx_v.at[b], isems[b]).wait()

                descs.append([
                    pltpu.async_copy(
                        table_hbm.at[idx_v.at[b, j]],
                        rows_v.at[b, pl.ds(j * GSUB, GSUB)],
                        gsems[b],
                    )
                    for j in range(NG)
                ])

            for b in range(NBUF):
                c = g * NBUF + b
                for d in descs[b]:
                    d.wait()

                @pl.when(g + 1 < n_pairs)
                def _():
                    pltpu.async_copy(
                        idx_slice(c + NBUF), idx_v.at[b], isems[b]
                    )

                pltpu.async_copy(rows_v.at[b], out_slice(c), osems[b])
            return carry

        lax.fori_loop(0, n_pairs, pair_body, 0)

        for b in range(NBUF):
            pltpu.make_async_copy(
                rows_v.at[b], out_slice(b), osems[b]
            ).wait()

    return k(x2d, weight)


def kernel(x, weight):
    B_, S_ = x.shape
    x2d = x.reshape(-1, GSUB).astype(jnp.int32)
    out = _embed_lookup(x2d, weight)
    return out.reshape(B_, S_, D)


# speedup vs baseline: 1.0747x; 1.0049x over previous
import functools

import jax
import jax.numpy as jnp
from jax import lax
from jax.experimental import pallas as pl
from jax.experimental.pallas import tpu as pltpu
from jax.experimental.pallas import tpu_sc as plsc

D = 64
NC = 2
NS = 16
NW = NC * NS
NR = 4
GHALVES = ((0, 104), (104, 96))
NBUF = 2


def kernel(x, weight):
    B_, S_ = x.shape
    assert S_ == sum(n for _, n in GHALVES)
    rows_per_w = B_ // NW
    n_chunks = rows_per_w // NR
    n_pairs = n_chunks // NBUF

    mesh = plsc.VectorSubcoreMesh(core_axis_name="c", subcore_axis_name="s")

    @functools.partial(
        pl.kernel,
        mesh=mesh,
        out_type=jax.ShapeDtypeStruct((B_, S_, D), jnp.float32),
        compiler_params=pltpu.CompilerParams(use_tc_tiling_on_sc=False),
        scratch_types=[
            pltpu.VMEM((NBUF, NR, S_), jnp.int32),
            pltpu.VMEM((NBUF, NR, S_, D), jnp.float32),
            pltpu.SemaphoreType.DMA,
            pltpu.SemaphoreType.DMA,
            pltpu.SemaphoreType.DMA,
            pltpu.SemaphoreType.DMA,
            pltpu.SemaphoreType.DMA,
            pltpu.SemaphoreType.DMA,
        ],
    )
    def k(x_hbm, table_hbm, out_hbm, idx_v, rows_v,
          isem0, isem1, gsem0, gsem1, osem0, osem1):
        isems = (isem0, isem1)
        gsems = (gsem0, gsem1)
        osems = (osem0, osem1)
        wid = lax.axis_index("s") * NC + lax.axis_index("c")
        row_base = wid * rows_per_w

        def x_slice(c):
            return x_hbm.at[pl.ds(row_base + c * NR, NR)]

        def out_slice(c):
            return out_hbm.at[pl.ds(row_base + c * NR, NR)]

        for b in range(NBUF):
            pltpu.async_copy(x_slice(b), idx_v.at[b], isems[b])

        def pair_body(g, carry):
            descs = []
            for b in range(NBUF):
                c = g * NBUF + b

                @pl.when(g > 0)
                def _():
                    pltpu.make_async_copy(
                        rows_v.at[b], out_slice(c), osems[b]
                    ).wait()

                pltpu.make_async_copy(x_slice(c), idx_v.at[b], isems[b]).wait()

                descs.append([
                    pltpu.async_copy(
                        table_hbm.at[idx_v.at[b, r, pl.ds(off, n)]],
                        rows_v.at[b, r, pl.ds(off, n)],
                        gsems[b],
                    )
                    for r in range(NR)
                    for off, n in GHALVES
                ])

            for b in range(NBUF):
                c = g * NBUF + b
                for dsc in descs[b]:
                    dsc.wait()

                @pl.when(g + 1 < n_pairs)
                def _():
                    pltpu.async_copy(
                        x_slice(c + NBUF), idx_v.at[b], isems[b]
                    )

                pltpu.async_copy(rows_v.at[b], out_slice(c), osems[b])
            return carry

        lax.fori_loop(0, n_pairs, pair_body, 0)

        for b in range(NBUF):
            pltpu.make_async_copy(
                rows_v.at[b], out_slice(b), osems[b]
            ).wait()

    return k(x, weight)
